# baseline (speedup 1.0000x reference)
import functools

import jax
import jax.numpy as jnp
from jax import lax
from jax.experimental import pallas as pl
from jax.experimental.pallas import tpu as pltpu
from jax.experimental.pallas import tpu_sc as plsc

N = 2_000_000
G = 161
GRID_FLAT = G * G * G
ASU = 1_000_000

NC = 2
NS = 16
NW = NC * NS

PADN = 1 << 21
PW = PADN // NW
CH = 16384
SEEN_PAD = 1 << 20
PER_TILE_SEEN = SEEN_PAD // NS
ZB = 8192

_PAD_F = G * G * G


def _linearize_body(h_ref, k_ref, l_ref, f_ref):
    f_ref[...] = h_ref[...] * (G * G) + k_ref[...] * G + l_ref[...]


def _merge_body(a_ref, b_ref, s_ref, o_ref):
    o_ref[...] = (a_ref[...] | b_ref[...] | s_ref[...]) != 0


def _sc_body(f_hbm, miller_hbm, out_hbm, seen2_hbm,
             idx_v, vals_v, ones_v, zer_v, seen_sp):
    c = lax.axis_index("c")
    s = lax.axis_index("s")
    wid = c * NS + s

    def fill_ones(i, carry):
        ones_v[pl.ds(i * 16, 16)] = jnp.full((16,), 1, jnp.int32)
        return carry

    lax.fori_loop(0, CH // 16, fill_ones, 0)

    def fill_zeros(i, carry):
        zer_v[pl.ds(i * 16, 16)] = jnp.zeros((16,), jnp.int32)
        return carry

    lax.fori_loop(0, ZB // 16, fill_zeros, 0)

    def zero_seen(i, carry):
        pltpu.sync_copy(zer_v,
                        seen_sp.at[pl.ds(s * PER_TILE_SEEN + i * ZB, ZB)])
        return carry

    lax.fori_loop(0, PER_TILE_SEEN // ZB, zero_seen, 0)
    plsc.subcore_barrier()

    def chunk(j, carry):
        base = wid * PW + j * CH
        pltpu.sync_copy(f_hbm.at[pl.ds(base, CH)], idx_v)
        pltpu.sync_copy(miller_hbm.at[idx_v], vals_v)
        pltpu.sync_copy(vals_v, out_hbm.at[pl.ds(base, CH)])
        pltpu.sync_copy(ones_v, seen_sp.at[vals_v])
        return carry

    lax.fori_loop(0, PW // CH, chunk, 0)
    plsc.subcore_barrier()

    pltpu.sync_copy(seen_sp.at[pl.ds(s * PER_TILE_SEEN, PER_TILE_SEEN)],
                    seen2_hbm.at[c, pl.ds(s * PER_TILE_SEEN, PER_TILE_SEEN)])


@functools.partial(
    pl.kernel,
    out_type=(
        jax.ShapeDtypeStruct((PADN,), jnp.int32),
        jax.ShapeDtypeStruct((NC, SEEN_PAD), jnp.int32),
    ),
    mesh=plsc.VectorSubcoreMesh(core_axis_name="c", subcore_axis_name="s"),
    scratch_types=[
        pltpu.VMEM((CH,), jnp.int32),
        pltpu.VMEM((CH,), jnp.int32),
        pltpu.VMEM((CH,), jnp.int32),
        pltpu.VMEM((ZB,), jnp.int32),
        pltpu.VMEM_SHARED((SEEN_PAD,), jnp.int32),
    ],
)
def _sc_gather_scatter(f_hbm, miller_hbm, out_hbm, seen2_hbm,
                       idx_v, vals_v, ones_v, zer_v, seen_sp):
    _sc_body(f_hbm, miller_hbm, out_hbm, seen2_hbm,
             idx_v, vals_v, ones_v, zer_v, seen_sp)


def kernel(hkl, miller_id, seen):
    hkl = hkl.astype(jnp.int32)
    pad = PADN - N
    h = jnp.concatenate([hkl[:, 0], jnp.full((pad,), G, jnp.int32)])
    k = jnp.concatenate([hkl[:, 1], jnp.zeros((pad,), jnp.int32)])
    l = jnp.concatenate([hkl[:, 2], jnp.zeros((pad,), jnp.int32)])
    shape2d = (PADN // 1024, 1024)
    h2, k2, l2 = h.reshape(shape2d), k.reshape(shape2d), l.reshape(shape2d)

    f2 = pl.pallas_call(
        _linearize_body,
        out_shape=jax.ShapeDtypeStruct(shape2d, jnp.int32),
        grid=(16,),
        in_specs=[pl.BlockSpec((shape2d[0] // 16, 1024), lambda i: (i, 0))] * 3,
        out_specs=pl.BlockSpec((shape2d[0] // 16, 1024), lambda i: (i, 0)),
    )(h2, k2, l2)
    f = f2.reshape(PADN)

    miller_p = jnp.concatenate(
        [miller_id.reshape(-1), jnp.full((7,), ASU, jnp.int32)])

    out_p, seen2 = _sc_gather_scatter(f, miller_p)

    seen32 = jnp.concatenate(
        [seen.astype(jnp.int32), jnp.zeros((SEEN_PAD - ASU,), jnp.int32)])
    mshape = (SEEN_PAD // 1024, 1024)
    merged = pl.pallas_call(
        _merge_body,
        out_shape=jax.ShapeDtypeStruct(mshape, jnp.bool_),
        grid=(8,),
        in_specs=[pl.BlockSpec((mshape[0] // 8, 1024), lambda i: (i, 0))] * 3,
        out_specs=pl.BlockSpec((mshape[0] // 8, 1024), lambda i: (i, 0)),
    )(seen2[0].reshape(mshape), seen2[1].reshape(mshape),
      seen32.reshape(mshape))

    return out_p[:N], merged.reshape(-1)[:ASU]


# speedup vs baseline: 1.0002x; 1.0002x over previous
import functools

import jax
import jax.numpy as jnp
from jax import lax
from jax.experimental import pallas as pl
from jax.experimental.pallas import tpu as pltpu
from jax.experimental.pallas import tpu_sc as plsc

N = 2_000_000
G = 161
GRID_FLAT = G * G * G
ASU = 1_000_000

NC = 2
NS = 16
NW = NC * NS

PADN = 1 << 21
PW = PADN // NW
CH = 16384
KSUB = 8
SUB = CH // KSUB
SEEN_PAD = 1 << 20
PER_TILE_SEEN = SEEN_PAD // NS
ZB = 8192

_PAD_F = G * G * G


def _linearize_body(h_ref, k_ref, l_ref, f_ref):
    f_ref[...] = h_ref[...] * (G * G) + k_ref[...] * G + l_ref[...]


def _merge_body(a_ref, b_ref, s_ref, o_ref):
    o_ref[...] = (a_ref[...] | b_ref[...] | s_ref[...]) != 0


def _sc_body(f_hbm, miller_hbm, out_hbm, seen2_hbm,
             idx_v, vals_v, ones_v, zer_v, seen_sp, gsem):
    c = lax.axis_index("c")
    s = lax.axis_index("s")
    wid = c * NS + s

    def fill_ones(i, carry):
        ones_v[pl.ds(i * 16, 16)] = jnp.full((16,), 1, jnp.int32)
        return carry

    lax.fori_loop(0, CH // 16, fill_ones, 0)

    def fill_zeros(i, carry):
        zer_v[pl.ds(i * 16, 16)] = jnp.zeros((16,), jnp.int32)
        return carry

    lax.fori_loop(0, ZB // 16, fill_zeros, 0)

    def zero_seen(i, carry):
        pltpu.sync_copy(zer_v,
                        seen_sp.at[pl.ds(s * PER_TILE_SEEN + i * ZB, ZB)])
        return carry

    lax.fori_loop(0, PER_TILE_SEEN // ZB, zero_seen, 0)
    plsc.subcore_barrier()

    def chunk(j, carry):
        base = wid * PW + j * CH
        pltpu.sync_copy(f_hbm.at[pl.ds(base, CH)], idx_v)
        descs = [
            pltpu.async_copy(
                miller_hbm.at[idx_v.at[pl.ds(t * SUB, SUB)]],
                vals_v.at[pl.ds(t * SUB, SUB)], gsem)
            for t in range(KSUB)
        ]
        for d in descs:
            d.wait()
        pltpu.sync_copy(vals_v, out_hbm.at[pl.ds(base, CH)])
        pltpu.sync_copy(ones_v, seen_sp.at[vals_v])
        return carry

    lax.fori_loop(0, PW // CH, chunk, 0)
    plsc.subcore_barrier()

    pltpu.sync_copy(seen_sp.at[pl.ds(s * PER_TILE_SEEN, PER_TILE_SEEN)],
                    seen2_hbm.at[c, pl.ds(s * PER_TILE_SEEN, PER_TILE_SEEN)])


@functools.partial(
    pl.kernel,
    out_type=(
        jax.ShapeDtypeStruct((PADN,), jnp.int32),
        jax.ShapeDtypeStruct((NC, SEEN_PAD), jnp.int32),
    ),
    mesh=plsc.VectorSubcoreMesh(core_axis_name="c", subcore_axis_name="s"),
    scratch_types=[
        pltpu.VMEM((CH,), jnp.int32),
        pltpu.VMEM((CH,), jnp.int32),
        pltpu.VMEM((CH,), jnp.int32),
        pltpu.VMEM((ZB,), jnp.int32),
        pltpu.VMEM_SHARED((SEEN_PAD,), jnp.int32),
        pltpu.SemaphoreType.DMA,
    ],
)
def _sc_gather_scatter(f_hbm, miller_hbm, out_hbm, seen2_hbm,
                       idx_v, vals_v, ones_v, zer_v, seen_sp, gsem):
    _sc_body(f_hbm, miller_hbm, out_hbm, seen2_hbm,
             idx_v, vals_v, ones_v, zer_v, seen_sp, gsem)


def kernel(hkl, miller_id, seen):
    hkl = hkl.astype(jnp.int32)
    pad = PADN - N
    h = jnp.concatenate([hkl[:, 0], jnp.full((pad,), G, jnp.int32)])
    k = jnp.concatenate([hkl[:, 1], jnp.zeros((pad,), jnp.int32)])
    l = jnp.concatenate([hkl[:, 2], jnp.zeros((pad,), jnp.int32)])
    shape2d = (PADN // 1024, 1024)
    h2, k2, l2 = h.reshape(shape2d), k.reshape(shape2d), l.reshape(shape2d)

    f2 = pl.pallas_call(
        _linearize_body,
        out_shape=jax.ShapeDtypeStruct(shape2d, jnp.int32),
        grid=(16,),
        in_specs=[pl.BlockSpec((shape2d[0] // 16, 1024), lambda i: (i, 0))] * 3,
        out_specs=pl.BlockSpec((shape2d[0] // 16, 1024), lambda i: (i, 0)),
    )(h2, k2, l2)
    f = f2.reshape(PADN)

    miller_p = jnp.concatenate(
        [miller_id.reshape(-1), jnp.full((7,), ASU, jnp.int32)])

    out_p, seen2 = _sc_gather_scatter(f, miller_p)

    seen32 = jnp.concatenate(
        [seen.astype(jnp.int32), jnp.zeros((SEEN_PAD - ASU,), jnp.int32)])
    mshape = (SEEN_PAD // 1024, 1024)
    merged = pl.pallas_call(
        _merge_body,
        out_shape=jax.ShapeDtypeStruct(mshape, jnp.bool_),
        grid=(8,),
        in_specs=[pl.BlockSpec((mshape[0] // 8, 1024), lambda i: (i, 0))] * 3,
        out_specs=pl.BlockSpec((mshape[0] // 8, 1024), lambda i: (i, 0)),
    )(seen2[0].reshape(mshape), seen2[1].reshape(mshape),
      seen32.reshape(mshape))

    return out_p[:N], merged.reshape(-1)[:ASU]


# speedup vs baseline: 1.1931x; 1.1929x over previous
import functools

import jax
import jax.numpy as jnp
from jax import lax
from jax.experimental import pallas as pl
from jax.experimental.pallas import tpu as pltpu
from jax.experimental.pallas import tpu_sc as plsc

N = 2_000_000
G = 161
GG = G * G
ASU = 1_000_000

NC = 2
NS = 16
NW = NC * NS

CH = 8192
NCHUNK = 8
SEEN_PAD = 1 << 20
PT_SEEN = SEEN_PAD // NS


def _merge_body(a_ref, b_ref, s_ref, o_ref):
    o_ref[...] = (a_ref[...] | b_ref[...] | s_ref[...]) != 0


def _sc_body(hklt_hbm, miller_hbm, out_hbm, seen2_hbm,
             h_v, k_v, l_v, idx_v, vals_v, ones_v, tsp):
    c = lax.axis_index("c")
    s = lax.axis_index("s")
    wid = c * NS + s

    def fill_ones(i, carry):
        ones_v[pl.ds(i * 16, 16)] = jnp.full((16,), 1, jnp.int32)
        return carry

    lax.fori_loop(0, CH // 16, fill_ones, 0)

    b0 = wid * 62_500 + 4 * (wid & 1)
    e0 = (wid + 1) * 62_500 + 4 * ((wid + 1) & 1)

    def do_chunk(t, base):
        base = pl.multiple_of(base, 8)
        pltpu.sync_copy(hklt_hbm.at[pl.ds(base, CH)], h_v)
        pltpu.sync_copy(hklt_hbm.at[pl.ds(base + N, CH)], k_v)
        pltpu.sync_copy(hklt_hbm.at[pl.ds(base + 2 * N, CH)], l_v)

        def lin(g, carry):
            sl = pl.ds(g * 16, 16)
            idx_v[sl] = h_v[sl] * GG + k_v[sl] * G + l_v[sl]
            return carry

        lax.fori_loop(0, CH // 16, lin, 0)
        pltpu.sync_copy(miller_hbm.at[idx_v], vals_v)
        pltpu.sync_copy(vals_v, out_hbm.at[pl.ds(base, CH)])

    for t in range(NCHUNK - 1):
        do_chunk(t, b0 + t * CH)
    do_chunk(NCHUNK - 1, e0 - CH)

    plsc.subcore_barrier()

    def fill_zeros(i, carry):
        ones_v[pl.ds(i * 16, 16)] = jnp.zeros((16,), jnp.int32)
        return carry

    lax.fori_loop(0, CH // 16, fill_zeros, 0)

    def zero_seen(i, carry):
        zo = pl.multiple_of(s * PT_SEEN + i * CH, 8)
        pltpu.sync_copy(ones_v, tsp.at[pl.ds(zo, CH)])
        return carry

    lax.fori_loop(0, PT_SEEN // CH, zero_seen, 0)

    lax.fori_loop(0, CH // 16, fill_ones, 0)
    plsc.subcore_barrier()

    def scat_chunk(t, base):
        base = pl.multiple_of(base, 8)
        pltpu.sync_copy(out_hbm.at[pl.ds(base, CH)], vals_v)
        pltpu.sync_copy(ones_v, tsp.at[vals_v])

    for t in range(NCHUNK - 1):
        scat_chunk(t, b0 + t * CH)
    scat_chunk(NCHUNK - 1, e0 - CH)

    plsc.subcore_barrier()
    po = pl.multiple_of(s * PT_SEEN, 8)
    pltpu.sync_copy(tsp.at[pl.ds(po, PT_SEEN)],
                    seen2_hbm.at[c, pl.ds(po, PT_SEEN)])


@functools.partial(
    pl.kernel,
    out_type=(
        jax.ShapeDtypeStruct((N,), jnp.int32),
        jax.ShapeDtypeStruct((NC, SEEN_PAD), jnp.int32),
    ),
    mesh=plsc.VectorSubcoreMesh(core_axis_name="c", subcore_axis_name="s"),
    scratch_types=[
        pltpu.VMEM((CH,), jnp.int32),
        pltpu.VMEM((CH,), jnp.int32),
        pltpu.VMEM((CH,), jnp.int32),
        pltpu.VMEM((CH,), jnp.int32),
        pltpu.VMEM((CH,), jnp.int32),
        pltpu.VMEM((CH,), jnp.int32),
        pltpu.VMEM_SHARED((SEEN_PAD,), jnp.int32),
    ],
)
def _sc_gather_scatter(hklt_hbm, miller_hbm, out_hbm, seen2_hbm,
                       h_v, k_v, l_v, idx_v, vals_v, ones_v, tsp):
    _sc_body(hklt_hbm, miller_hbm, out_hbm, seen2_hbm,
             h_v, k_v, l_v, idx_v, vals_v, ones_v, tsp)


def kernel(hkl, miller_id, seen):
    hklt = hkl.astype(jnp.int32).T.reshape(-1)
    miller_flat = miller_id.reshape(-1)

    out, seen2 = _sc_gather_scatter(hklt, miller_flat)

    seen32 = jnp.concatenate(
        [seen.astype(jnp.int32), jnp.zeros((SEEN_PAD - ASU,), jnp.int32)])
    mshape = (SEEN_PAD // 1024, 1024)
    merged = pl.pallas_call(
        _merge_body,
        out_shape=jax.ShapeDtypeStruct(mshape, jnp.bool_),
        grid=(8,),
        in_specs=[pl.BlockSpec((mshape[0] // 8, 1024), lambda i: (i, 0))] * 3,
        out_specs=pl.BlockSpec((mshape[0] // 8, 1024), lambda i: (i, 0)),
    )(seen2[0].reshape(mshape), seen2[1].reshape(mshape),
      seen32.reshape(mshape))

    return out, merged.reshape(-1)[:ASU]


# speedup vs baseline: 1.2986x; 1.0884x over previous
import functools

import jax
import jax.numpy as jnp
from jax import lax
from jax.experimental import pallas as pl
from jax.experimental.pallas import tpu as pltpu
from jax.experimental.pallas import tpu_sc as plsc

N = 2_000_000
G = 161
GG = G * G
ASU = 1_000_000

NC = 2
NS = 16
NW = NC * NS

CH = 4096
NCHUNK = 16
SEEN_PAD = 1 << 20
PT_SEEN = SEEN_PAD // NS


def _merge_body(a_ref, b_ref, s_ref, o_ref):
    o_ref[...] = (a_ref[...] | b_ref[...] | s_ref[...]) != 0


def _sc_body(hklt_hbm, miller_hbm, out_hbm, seen2_hbm,
             h0, h1, k0, k1, l0, l1, i0, i1, v0, v1, ones_v, tsp,
             sh0, sh1, sg0, sg1, so0, so1, ss0, ss1, sz):
    c = lax.axis_index("c")
    s = lax.axis_index("s")
    wid = c * NS + s

    h_v, k_v, l_v = (h0, h1), (k0, k1), (l0, l1)
    idx_v, vals_v = (i0, i1), (v0, v1)
    sem_h, sem_g, sem_o, sem_s = (sh0, sh1), (sg0, sg1), (so0, so1), (ss0, ss1)

    def fill_zeros(i, carry):
        i0[pl.ds(i * 16, 16)] = jnp.zeros((16,), jnp.int32)
        return carry

    lax.fori_loop(0, CH // 16, fill_zeros, 0)

    zdescs = []
    for i in range(PT_SEEN // CH):
        zdescs.append(
            pltpu.async_copy(i0, tsp.at[pl.ds(s * PT_SEEN + i * CH, CH)], sz))

    def fill_ones(i, carry):
        ones_v[pl.ds(i * 16, 16)] = jnp.full((16,), 1, jnp.int32)
        return carry

    lax.fori_loop(0, CH // 16, fill_ones, 0)

    b0 = wid * 62_500 + 4 * (wid & 1)
    e0 = (wid + 1) * 62_500 + 4 * ((wid + 1) & 1)
    bases = [pl.multiple_of(b0 + t * CH, 8) for t in range(NCHUNK - 1)]
    bases.append(pl.multiple_of(e0 - CH, 8))

    hkl_descs = [None, None]

    def prefetch(j, b):
        base = bases[j]
        hkl_descs[b] = (
            pltpu.async_copy(hklt_hbm.at[pl.ds(base, CH)], h_v[b], sem_h[b]),
            pltpu.async_copy(hklt_hbm.at[pl.ds(base + N, CH)], k_v[b],
                             sem_h[b]),
            pltpu.async_copy(hklt_hbm.at[pl.ds(base + 2 * N, CH)], l_v[b],
                             sem_h[b]),
        )

    def lin(b):
        def body(g, carry):
            sl = pl.ds(g * 16, 16)
            idx_v[b][sl] = h_v[b][sl] * GG + k_v[b][sl] * G + l_v[b][sl]
            return carry

        lax.fori_loop(0, CH // 16, body, 0)

    prefetch(0, 0)
    prefetch(1, 1)

    for d in zdescs:
        d.wait()
    plsc.subcore_barrier()

    for d in hkl_descs[0]:
        d.wait()
    lin(0)

    g_desc = [None, None]
    o_desc = [None, None]
    s_desc = [None, None]
    for j in range(NCHUNK):
        b = j & 1
        if o_desc[b] is not None:
            o_desc[b].wait()
            s_desc[b].wait()
        g_desc[b] = pltpu.async_copy(miller_hbm.at[idx_v[b]], vals_v[b],
                                     sem_g[b])
        if j + 2 < NCHUNK:
            prefetch(j + 2, b)
        if j + 1 < NCHUNK:
            for d in hkl_descs[b ^ 1]:
                d.wait()
            lin(b ^ 1)
        g_desc[b].wait()
        o_desc[b] = pltpu.async_copy(vals_v[b], out_hbm.at[pl.ds(bases[j], CH)],
                                     sem_o[b])
        s_desc[b] = pltpu.async_copy(ones_v, tsp.at[vals_v[b]], sem_s[b])

    for b in (0, 1):
        o_desc[b].wait()
        s_desc[b].wait()

    plsc.subcore_barrier()
    po = pl.multiple_of(s * PT_SEEN, 8)
    pltpu.sync_copy(tsp.at[pl.ds(po, PT_SEEN)],
                    seen2_hbm.at[c, pl.ds(po, PT_SEEN)])


@functools.partial(
    pl.kernel,
    out_type=(
        jax.ShapeDtypeStruct((N,), jnp.int32),
        jax.ShapeDtypeStruct((NC, SEEN_PAD), jnp.int32),
    ),
    mesh=plsc.VectorSubcoreMesh(core_axis_name="c", subcore_axis_name="s"),
    scratch_types=(
        [pltpu.VMEM((CH,), jnp.int32) for _ in range(11)]
        + [pltpu.VMEM_SHARED((SEEN_PAD,), jnp.int32)]
        + [pltpu.SemaphoreType.DMA for _ in range(9)]
    ),
)
def _sc_gather_scatter(hklt_hbm, miller_hbm, out_hbm, seen2_hbm, *scratch):
    _sc_body(hklt_hbm, miller_hbm, out_hbm, seen2_hbm, *scratch)


def kernel(hkl, miller_id, seen):
    hklt = hkl.astype(jnp.int32).T.reshape(-1)
    miller_flat = miller_id.reshape(-1)

    out, seen2 = _sc_gather_scatter(hklt, miller_flat)

    seen32 = jnp.concatenate(
        [seen.astype(jnp.int32), jnp.zeros((SEEN_PAD - ASU,), jnp.int32)])
    mshape = (SEEN_PAD // 1024, 1024)
    merged = pl.pallas_call(
        _merge_body,
        out_shape=jax.ShapeDtypeStruct(mshape, jnp.bool_),
        grid=(8,),
        in_specs=[pl.BlockSpec((mshape[0] // 8, 1024), lambda i: (i, 0))] * 3,
        out_specs=pl.BlockSpec((mshape[0] // 8, 1024), lambda i: (i, 0)),
    )(seen2[0].reshape(mshape), seen2[1].reshape(mshape),
      seen32.reshape(mshape))

    return out, merged.reshape(-1)[:ASU]
